# baseline (speedup 1.0000x reference)
import numpy as np

import jax
import jax.numpy as jnp
from jax import lax
from jax.experimental import pallas as pl
from jax.experimental.pallas import tpu as pltpu
from jax.experimental.pallas import tpu_sc as plsc

TOP_K = 30
NUM_RBF = 16
POS_EMB_DIMS = 16
SEQ_NEIGHBORS = 30

_ROWS = 256
_FROWS = 256


_STRIP = 8


def _topk_body(q_ref, kt_ref, d_ref, i_ref, g_ref):
    q = q_ref[0]
    R = q.shape[0]
    L = kt_ref.shape[2]
    kx = kt_ref[0, 0:1, :]
    ky = kt_ref[0, 1:2, :]
    kz = kt_ref[0, 2:3, :]
    S = _STRIP
    col = lax.broadcasted_iota(jnp.int32, (S, L), 1)
    row = lax.broadcasted_iota(jnp.int32, (S, L), 0)
    base = pl.program_id(1) * R

    for s in range(R // S):
        qs = q[S * s:S * s + S]
        dx = qs[:, 0:1] - kx
        dy = qs[:, 1:2] - ky
        dz = qs[:, 2:3] - kz
        ss = dx * dx + dy * dy + dz * dz
        D = jnp.sqrt(ss + 1e-6)
        D = jnp.where(col == row + (base + S * s), jnp.float32(10000.0), D)

        vals = []
        idxs = []
        for _ in range(TOP_K):
            m = jnp.min(D, axis=1, keepdims=True)
            idx = jnp.min(jnp.where(D == m, col, L), axis=1, keepdims=True)
            D = jnp.where(col == idx, jnp.float32(jnp.inf), D)
            vals.append(m)
            idxs.append(idx)
        loc = jnp.concatenate(idxs, axis=1)
        d_ref[0, S * s:S * s + S] = jnp.concatenate(vals, axis=1)
        i_ref[0, S * s:S * s + S] = loc
        g_ref[0, S * s:S * s + S] = loc + pl.program_id(0) * L


def _dist_topk(Xc):
    B, L, _ = Xc.shape
    Xct = jnp.swapaxes(Xc, 1, 2)
    grid = (B, L // _ROWS)
    return pl.pallas_call(
        _topk_body,
        grid=grid,
        in_specs=[
            pl.BlockSpec((1, _ROWS, 3), lambda b, i: (b, i, 0)),
            pl.BlockSpec((1, 3, L), lambda b, i: (b, 0, 0)),
        ],
        out_specs=[
            pl.BlockSpec((1, _ROWS, TOP_K), lambda b, i: (b, i, 0)),
            pl.BlockSpec((1, _ROWS, TOP_K), lambda b, i: (b, i, 0)),
            pl.BlockSpec((1, _ROWS, TOP_K), lambda b, i: (b, i, 0)),
        ],
        out_shape=[
            jax.ShapeDtypeStruct((B, L, TOP_K), jnp.float32),
            jax.ShapeDtypeStruct((B, L, TOP_K), jnp.int32),
            jax.ShapeDtypeStruct((B, L, TOP_K), jnp.int32),
        ],
    )(Xc, Xct)


def _shift_up(v):
    return jnp.concatenate([v[1:], jnp.zeros((1, 1), v.dtype)], axis=0)


def _shift_down(v):
    return jnp.concatenate([jnp.zeros((1, 1), v.dtype), v[:-1]], axis=0)


def _norm3(v, eps2=1e-24):
    n2 = v[0] * v[0] + v[1] * v[1] + v[2] * v[2]
    inv = 1.0 / jnp.sqrt(jnp.clip(n2, eps2, None))
    return [v[0] * inv, v[1] * inv, v[2] * inv]


def _cross3(a, b):
    return [a[1] * b[2] - a[2] * b[1],
            a[2] * b[0] - a[0] * b[2],
            a[0] * b[1] - a[1] * b[0]]


def _dot3(a, b):
    return a[0] * b[0] + a[1] * b[1] + a[2] * b[2]


def _dihedral_phase(a, b, c, valid, eps=1e-7):
    n2v = _norm3(_cross3(a, b))
    n1v = _norm3(_cross3(b, c))
    cosd = jnp.clip(_dot3(n2v, n1v), -1.0 + eps, 1.0 - eps)
    sgn = jnp.sign(_dot3(a, n1v))
    cosout = jnp.where(valid, cosd, 1.0)
    sinout = jnp.where(valid, sgn * jnp.sqrt(1.0 - cosd * cosd), 0.0)
    return cosout, sinout


def _frames_body(x_ref, t_ref, v_ref):
    x = x_ref[0]
    Lr = x.shape[0]
    A = [[x[:, 3 * a + c:3 * a + c + 1] for c in range(3)] for a in range(3)]
    ri = lax.broadcasted_iota(jnp.int32, (Lr, 1), 0)

    u0 = _norm3([A[1][c] - A[0][c] for c in range(3)])
    u1 = _norm3([A[2][c] - A[1][c] for c in range(3)])
    u2 = _norm3([_shift_up(A[0][c]) - A[2][c] for c in range(3)])
    u2m1 = [_shift_down(u2[c]) for c in range(3)]
    u0p1 = [_shift_up(u0[c]) for c in range(3)]

    cos0, sin0 = _dihedral_phase(u2m1, u0, u1, ri >= 1)
    cos1, sin1 = _dihedral_phase(u0, u1, u2, ri <= Lr - 2)
    cos2, sin2 = _dihedral_phase(u1, u2, u0p1, ri <= Lr - 2)
    v_ref[0] = jnp.concatenate([cos0, cos1, cos2, sin0, sin1, sin2], axis=1)

    Ca = A[1]
    Uc = _norm3([_shift_up(Ca[c]) - Ca[c] for c in range(3)])
    Um1 = [_shift_down(Uc[c]) for c in range(3)]
    o1 = _norm3([Um1[c] - Uc[c] for c in range(3)])
    n2v = _norm3(_cross3(Um1, Uc))
    r3 = _cross3(o1, n2v)
    fvalid = (ri >= 1) & (ri <= Lr - 3)
    cols = []
    for p in (o1, n2v, r3):
        cols.extend(jnp.where(fvalid, p[c], 0.0) for c in range(3))
    cols.extend(Ca)
    t_ref[0] = jnp.concatenate(cols, axis=1)


def _frames_dihedrals(Xr):
    B, L, _ = Xr.shape
    return pl.pallas_call(
        _frames_body,
        grid=(B,),
        in_specs=[pl.BlockSpec((1, L, 12), lambda b: (b, 0, 0))],
        out_specs=[
            pl.BlockSpec((1, L, 12), lambda b: (b, 0, 0)),
            pl.BlockSpec((1, L, 6), lambda b: (b, 0, 0)),
        ],
        out_shape=[
            jax.ShapeDtypeStruct((B, L, 12), jnp.float32),
            jax.ShapeDtypeStruct((B, L, 6), jnp.float32),
        ],
    )(Xr)


_NCOMP = 12


def _sc_gather(tableT, idx):
    Btot = idx.shape[0]
    V = tableT.shape[1]
    info = plsc.get_sparse_core_info()
    NC, NS = info.num_cores, info.num_subcores
    NW = NC * NS
    b_per_w = Btot // NW
    chunk = 1920
    nchunks = b_per_w // chunk
    mesh = plsc.VectorSubcoreMesh(core_axis_name="c", subcore_axis_name="s")

    @pl.kernel(
        mesh=mesh,
        compiler_params=pltpu.CompilerParams(needs_layout_passes=False),
        out_type=jax.ShapeDtypeStruct((_NCOMP, Btot), jnp.float32),
        scratch_types=(
            [pltpu.VMEM((V,), jnp.float32) for _ in range(_NCOMP)]
            + [pltpu.VMEM((chunk,), jnp.int32)]
            + [pltpu.VMEM((_NCOMP, chunk), jnp.float32)]
            + [pltpu.SemaphoreType.DMA]
        ),
    )
    def gk(table_hbm, idx_hbm, out_hbm, *bufs):
        tab = bufs[:_NCOMP]
        idx_v = bufs[_NCOMP]
        outb = bufs[_NCOMP + 1]
        sem = bufs[_NCOMP + 2]
        wid = lax.axis_index("s") * NC + lax.axis_index("c")
        copies = [pltpu.async_copy(table_hbm.at[c], tab[c], sem)
                  for c in range(_NCOMP)]
        for cp in copies:
            cp.wait()
        base_w = wid * b_per_w
        for t in range(nchunks):
            base = base_w + t * chunk
            pltpu.sync_copy(idx_hbm.at[pl.ds(base, chunk)], idx_v)

            def grp(g, carry):
                for u in range(8):
                    o = g * 128 + u * 16
                    iv = idx_v[pl.ds(o, 16)]
                    for c in range(_NCOMP):
                        outb[c, pl.ds(o, 16)] = plsc.load_gather(tab[c], [iv])
                return carry

            lax.fori_loop(0, chunk // 128, grp, 0)
            pltpu.async_copy(outb, out_hbm.at[:, pl.ds(base, chunk)], sem).wait()

    return gk(tableT, idx)


def _features_body(gn_ref, gq_ref, d_ref, i_ref, e_ref):
    Rr = d_ref.shape[0]
    N = [gn_ref[c] for c in range(_NCOMP)]
    Q = [gq_ref[c] for c in range(_NCOMP)]
    Dv = d_ref[...]
    idxf = i_ref[...].astype(jnp.float32)

    r_io = lax.broadcasted_iota(jnp.int32, (Rr, 128), 0)
    l_io = lax.broadcasted_iota(jnp.int32, (Rr, 128), 1)
    e_in_b = (pl.program_id(1) * Rr + r_io) * 128 + l_io
    i_loc = (e_in_b // TOP_K).astype(jnp.float32)

    out = [None] * 39

    d = idxf - i_loc
    d = jnp.where(jnp.abs(d) > SEQ_NEIGHBORS, 0.0, d)
    emask = (d != 0.0).astype(jnp.float32)
    c1 = np.float32(6.28125)
    c2 = np.float32(2.0 * np.pi - 6.28125)
    for j in range(POS_EMB_DIMS // 2):
        freq = np.float32(np.exp(2 * j * (-np.log(10000.0) / POS_EMB_DIMS)))
        ang = d * freq
        k = jnp.floor(ang * np.float32(1.0 / (2.0 * np.pi)) + 0.5)
        ang = (ang - k * c1) - k * c2
        out[j] = jnp.cos(ang) * emask
        out[8 + j] = jnp.sin(ang) * emask

    sigma = np.float32(20.0 / NUM_RBF)
    mus = np.linspace(0.0, 20.0, NUM_RBF, dtype=np.float32)
    for j in range(NUM_RBF):
        t = (Dv - mus[j]) * np.float32(1.0 / sigma)
        out[16 + j] = jnp.exp(-(t * t))

    def _b(x):
        return x.astype(jnp.bfloat16).astype(jnp.float32)

    dxn = [_b(N[9 + c] - Q[9 + c]) for c in range(3)]
    Qb = [_b(Q[c]) for c in range(9)]
    Nb = [_b(N[c]) for c in range(9)]
    du = _norm3([(Qb[3 * r + 0] * dxn[0] + Qb[3 * r + 1] * dxn[1]) + Qb[3 * r + 2] * dxn[2]
                 for r in range(3)])
    for c in range(3):
        out[32 + c] = du[c]

    R = [[(Qb[0 + r] * Nb[0 + c] + Qb[3 + r] * Nb[3 + c]) + Qb[6 + r] * Nb[6 + c]
          for c in range(3)] for r in range(3)]
    mag_args = [R[0][0] - R[1][1] - R[2][2],
                -R[0][0] + R[1][1] - R[2][2],
                -R[0][0] - R[1][1] + R[2][2]]
    sign_args = [R[2][1] - R[1][2], R[0][2] - R[2][0], R[1][0] - R[0][1]]
    q = [jnp.sign(sign_args[c]) * (0.5 * jnp.sqrt(jnp.abs(1.0 + mag_args[c])))
         for c in range(3)]
    trace = R[0][0] + R[1][1] + R[2][2]
    q.append(jnp.sqrt(jax.nn.relu(1.0 + trace)) * 0.5)
    qn2 = q[0] * q[0] + q[1] * q[1] + q[2] * q[2] + q[3] * q[3]
    qinv = 1.0 / jnp.sqrt(jnp.clip(qn2, 1e-24, None))
    for c in range(4):
        out[35 + c] = q[c] * qinv

    for c in range(39):
        e_ref[c] = out[c]


def _features(Gn, Gq, Dnb, Eidx):
    B, L, K = Dnb.shape
    rows_b = L * K // 128
    S = 2
    sub = rows_b // S
    gn = Gn.reshape(_NCOMP, B * rows_b, 128)
    gq = Gq.reshape(_NCOMP, B * rows_b, 128)
    dnb = Dnb.reshape(B * rows_b, 128)
    eidx = Eidx.reshape(B * rows_b, 128)
    return pl.pallas_call(
        _features_body,
        grid=(B, S),
        in_specs=[
            pl.BlockSpec((_NCOMP, sub, 128), lambda b, s: (0, b * 2 + s, 0)),
            pl.BlockSpec((_NCOMP, sub, 128), lambda b, s: (0, b * 2 + s, 0)),
            pl.BlockSpec((sub, 128), lambda b, s: (b * 2 + s, 0)),
            pl.BlockSpec((sub, 128), lambda b, s: (b * 2 + s, 0)),
        ],
        out_specs=pl.BlockSpec((39, sub, 128), lambda b, s: (0, b * 2 + s, 0)),
        out_shape=jax.ShapeDtypeStruct((39, B * rows_b, 128), jnp.float32),
    )(gn, gq, dnb, eidx)


def kernel(X, mask):
    B, N = X.shape[0], X.shape[2]
    K = TOP_K
    Xr = X.reshape(B, N, 12)
    Xc = X[:, 0, :, 1, :]

    D_neighbors, E_idx, G_idx = _dist_topk(Xc)
    table, V = _frames_dihedrals(Xr)

    tableT = table.reshape(B * N, _NCOMP).T
    idx_all = jnp.concatenate([
        G_idx.reshape(-1),
        jnp.repeat(jnp.arange(B * N, dtype=jnp.int32), K),
    ])
    G = _sc_gather(tableT, idx_all)
    half = B * N * K
    Gn = G[:, :half]
    Gq = G[:, half:]
    Eplanes = _features(Gn, Gq, D_neighbors, E_idx)
    E = Eplanes.reshape(39, half).T.reshape(B, N, K, 39)
    return (V, E, E_idx)


# speedup vs baseline: 1.2656x; 1.2656x over previous
import numpy as np

import jax
import jax.numpy as jnp
from jax import lax
from jax.experimental import pallas as pl
from jax.experimental.pallas import tpu as pltpu
from jax.experimental.pallas import tpu_sc as plsc

TOP_K = 30
NUM_RBF = 16
POS_EMB_DIMS = 16
SEQ_NEIGHBORS = 30

_ROWS = 256
_FROWS = 256


_STRIP = 8


def _topk_body(q_ref, kt_ref, d_ref, i_ref, g_ref):
    q = q_ref[0]
    R = q.shape[0]
    L = kt_ref.shape[2]
    kx = kt_ref[0, 0:1, :]
    ky = kt_ref[0, 1:2, :]
    kz = kt_ref[0, 2:3, :]
    S = _STRIP
    col = lax.broadcasted_iota(jnp.int32, (S, L), 1)
    row = lax.broadcasted_iota(jnp.int32, (S, L), 0)
    base = pl.program_id(1) * R

    for s in range(R // S):
        qs = q[S * s:S * s + S]
        dx = qs[:, 0:1] - kx
        dy = qs[:, 1:2] - ky
        dz = qs[:, 2:3] - kz
        ss = dx * dx + dy * dy + dz * dz
        D = jnp.sqrt(ss + 1e-6)
        D = jnp.where(col == row + (base + S * s), jnp.float32(10000.0), D)

        vals = []
        idxs = []
        for _ in range(TOP_K):
            m = jnp.min(D, axis=1, keepdims=True)
            idx = jnp.min(jnp.where(D == m, col, L), axis=1, keepdims=True)
            D = jnp.where(col == idx, jnp.float32(jnp.inf), D)
            vals.append(m)
            idxs.append(idx)
        loc = jnp.concatenate(idxs, axis=1)
        d_ref[0, S * s:S * s + S] = jnp.concatenate(vals, axis=1)
        i_ref[0, S * s:S * s + S] = loc
        g_ref[0, S * s:S * s + S] = loc + pl.program_id(0) * L


def _dist_topk(Xc):
    B, L, _ = Xc.shape
    Xct = jnp.swapaxes(Xc, 1, 2)
    grid = (B, L // _ROWS)
    return pl.pallas_call(
        _topk_body,
        grid=grid,
        in_specs=[
            pl.BlockSpec((1, _ROWS, 3), lambda b, i: (b, i, 0)),
            pl.BlockSpec((1, 3, L), lambda b, i: (b, 0, 0)),
        ],
        out_specs=[
            pl.BlockSpec((1, _ROWS, TOP_K), lambda b, i: (b, i, 0)),
            pl.BlockSpec((1, _ROWS, TOP_K), lambda b, i: (b, i, 0)),
            pl.BlockSpec((1, _ROWS, TOP_K), lambda b, i: (b, i, 0)),
        ],
        out_shape=[
            jax.ShapeDtypeStruct((B, L, TOP_K), jnp.float32),
            jax.ShapeDtypeStruct((B, L, TOP_K), jnp.int32),
            jax.ShapeDtypeStruct((B, L, TOP_K), jnp.int32),
        ],
    )(Xc, Xct)


def _shift_up(v):
    return jnp.concatenate([v[1:], jnp.zeros((1, 1), v.dtype)], axis=0)


def _shift_down(v):
    return jnp.concatenate([jnp.zeros((1, 1), v.dtype), v[:-1]], axis=0)


def _norm3(v, eps2=1e-24):
    n2 = v[0] * v[0] + v[1] * v[1] + v[2] * v[2]
    inv = 1.0 / jnp.sqrt(jnp.clip(n2, eps2, None))
    return [v[0] * inv, v[1] * inv, v[2] * inv]


def _cross3(a, b):
    return [a[1] * b[2] - a[2] * b[1],
            a[2] * b[0] - a[0] * b[2],
            a[0] * b[1] - a[1] * b[0]]


def _dot3(a, b):
    return a[0] * b[0] + a[1] * b[1] + a[2] * b[2]


def _dihedral_phase(a, b, c, valid, eps=1e-7):
    n2v = _norm3(_cross3(a, b))
    n1v = _norm3(_cross3(b, c))
    cosd = jnp.clip(_dot3(n2v, n1v), -1.0 + eps, 1.0 - eps)
    sgn = jnp.sign(_dot3(a, n1v))
    cosout = jnp.where(valid, cosd, 1.0)
    sinout = jnp.where(valid, sgn * jnp.sqrt(1.0 - cosd * cosd), 0.0)
    return cosout, sinout


def _table_body(x_ref, t_ref):
    x = x_ref[0]
    Lr = x.shape[0]
    Ca = [x[:, c:c + 1] for c in range(3)]
    ri = lax.broadcasted_iota(jnp.int32, (Lr, 1), 0)
    Uc = _norm3([_shift_up(Ca[c]) - Ca[c] for c in range(3)])
    Um1 = [_shift_down(Uc[c]) for c in range(3)]
    o1 = _norm3([Um1[c] - Uc[c] for c in range(3)])
    n2v = _norm3(_cross3(Um1, Uc))
    r3 = _cross3(o1, n2v)
    fvalid = (ri >= 1) & (ri <= Lr - 3)
    cols = []
    for p in (o1, n2v, r3):
        cols.extend(jnp.where(fvalid, p[c], 0.0) for c in range(3))
    cols.extend(Ca)
    t_ref[0] = jnp.concatenate(cols, axis=1)


def _table_kernel(Xc):
    B, L, _ = Xc.shape
    return pl.pallas_call(
        _table_body,
        grid=(B,),
        in_specs=[pl.BlockSpec((1, L, 3), lambda b: (b, 0, 0))],
        out_specs=pl.BlockSpec((1, L, 12), lambda b: (b, 0, 0)),
        out_shape=jax.ShapeDtypeStruct((B, L, 12), jnp.float32),
    )(Xc)


def _dihedrals_body(x_ref, v_ref):
    x = x_ref[0]
    Lr = x.shape[0]
    A = [[x[:, 3 * a + c:3 * a + c + 1] for c in range(3)] for a in range(3)]
    ri = lax.broadcasted_iota(jnp.int32, (Lr, 1), 0)
    u0 = _norm3([A[1][c] - A[0][c] for c in range(3)])
    u1 = _norm3([A[2][c] - A[1][c] for c in range(3)])
    u2 = _norm3([_shift_up(A[0][c]) - A[2][c] for c in range(3)])
    u2m1 = [_shift_down(u2[c]) for c in range(3)]
    u0p1 = [_shift_up(u0[c]) for c in range(3)]
    cos0, sin0 = _dihedral_phase(u2m1, u0, u1, ri >= 1)
    cos1, sin1 = _dihedral_phase(u0, u1, u2, ri <= Lr - 2)
    cos2, sin2 = _dihedral_phase(u1, u2, u0p1, ri <= Lr - 2)
    v_ref[0] = jnp.concatenate([cos0, cos1, cos2, sin0, sin1, sin2], axis=1)


def _dihedrals_kernel(Xr):
    B, L, _ = Xr.shape
    return pl.pallas_call(
        _dihedrals_body,
        grid=(B,),
        in_specs=[pl.BlockSpec((1, L, 12), lambda b: (b, 0, 0))],
        out_specs=pl.BlockSpec((1, L, 6), lambda b: (b, 0, 0)),
        out_shape=jax.ShapeDtypeStruct((B, L, 6), jnp.float32),
    )(Xr)


_NCOMP = 12


def _sc_gather(tableT, idx):
    Btot = idx.shape[0]
    V = tableT.shape[1]
    info = plsc.get_sparse_core_info()
    NC, NS = info.num_cores, info.num_subcores
    NW = NC * NS
    b_per_w = Btot // NW
    chunk = 1920
    nchunks = b_per_w // chunk
    mesh = plsc.VectorSubcoreMesh(core_axis_name="c", subcore_axis_name="s")

    @pl.kernel(
        mesh=mesh,
        compiler_params=pltpu.CompilerParams(needs_layout_passes=False),
        out_type=jax.ShapeDtypeStruct((_NCOMP, Btot), jnp.float32),
        scratch_types=(
            [pltpu.VMEM((V,), jnp.float32) for _ in range(_NCOMP)]
            + [pltpu.VMEM((chunk,), jnp.int32)]
            + [pltpu.VMEM((_NCOMP, chunk), jnp.float32)]
            + [pltpu.SemaphoreType.DMA]
        ),
    )
    def gk(table_hbm, idx_hbm, out_hbm, *bufs):
        tab = bufs[:_NCOMP]
        idx_v = bufs[_NCOMP]
        outb = bufs[_NCOMP + 1]
        sem = bufs[_NCOMP + 2]
        wid = lax.axis_index("s") * NC + lax.axis_index("c")
        copies = [pltpu.async_copy(table_hbm.at[c], tab[c], sem)
                  for c in range(_NCOMP)]
        for cp in copies:
            cp.wait()
        base_w = wid * b_per_w
        for t in range(nchunks):
            base = base_w + t * chunk
            pltpu.sync_copy(idx_hbm.at[pl.ds(base, chunk)], idx_v)

            def grp(g, carry):
                for u in range(8):
                    o = g * 128 + u * 16
                    iv = idx_v[pl.ds(o, 16)]
                    for c in range(_NCOMP):
                        outb[c, pl.ds(o, 16)] = plsc.load_gather(tab[c], [iv])
                return carry

            lax.fori_loop(0, chunk // 128, grp, 0)
            pltpu.async_copy(outb, out_hbm.at[:, pl.ds(base, chunk)], sem).wait()

    return gk(tableT, idx)


def _features_body(gn_ref, t_ref, d_ref, i_ref, e_ref):
    Rr = d_ref.shape[1]
    K = d_ref.shape[2]
    N = [gn_ref[c, 0] for c in range(_NCOMP)]
    Q = [t_ref[0, :, c:c + 1] for c in range(_NCOMP)]
    Dv = d_ref[0]
    idxf = i_ref[0].astype(jnp.float32)

    base = pl.program_id(1) * Rr
    i_loc = (base + lax.broadcasted_iota(jnp.int32, (Rr, 1), 0)).astype(jnp.float32)

    out = [None] * 39

    d = idxf - i_loc
    d = jnp.where(jnp.abs(d) > SEQ_NEIGHBORS, 0.0, d)
    emask = (d != 0.0).astype(jnp.float32)
    c1 = np.float32(6.28125)
    c2 = np.float32(2.0 * np.pi - 6.28125)
    for j in range(POS_EMB_DIMS // 2):
        freq = np.float32(np.exp(2 * j * (-np.log(10000.0) / POS_EMB_DIMS)))
        ang = d * freq
        k = jnp.floor(ang * np.float32(1.0 / (2.0 * np.pi)) + 0.5)
        ang = (ang - k * c1) - k * c2
        out[j] = jnp.cos(ang) * emask
        out[8 + j] = jnp.sin(ang) * emask

    sigma = np.float32(20.0 / NUM_RBF)
    mus = np.linspace(0.0, 20.0, NUM_RBF, dtype=np.float32)
    for j in range(NUM_RBF):
        t = (Dv - mus[j]) * np.float32(1.0 / sigma)
        out[16 + j] = jnp.exp(-(t * t))

    def _b(x):
        return x.astype(jnp.bfloat16).astype(jnp.float32)

    dxn = [_b(N[9 + c] - Q[9 + c]) for c in range(3)]
    Qb = [_b(Q[c]) for c in range(9)]
    Nb = [_b(N[c]) for c in range(9)]
    du = _norm3([(Qb[3 * r + 0] * dxn[0] + Qb[3 * r + 1] * dxn[1]) + Qb[3 * r + 2] * dxn[2]
                 for r in range(3)])
    for c in range(3):
        out[32 + c] = du[c]

    R = [[(Qb[0 + r] * Nb[0 + c] + Qb[3 + r] * Nb[3 + c]) + Qb[6 + r] * Nb[6 + c]
          for c in range(3)] for r in range(3)]
    mag_args = [R[0][0] - R[1][1] - R[2][2],
                -R[0][0] + R[1][1] - R[2][2],
                -R[0][0] - R[1][1] + R[2][2]]
    sign_args = [R[2][1] - R[1][2], R[0][2] - R[2][0], R[1][0] - R[0][1]]
    q = [jnp.sign(sign_args[c]) * (0.5 * jnp.sqrt(jnp.abs(1.0 + mag_args[c])))
         for c in range(3)]
    trace = R[0][0] + R[1][1] + R[2][2]
    q.append(jnp.sqrt(jax.nn.relu(1.0 + trace)) * 0.5)
    qn2 = q[0] * q[0] + q[1] * q[1] + q[2] * q[2] + q[3] * q[3]
    qinv = 1.0 / jnp.sqrt(jnp.clip(qn2, 1e-24, None))
    for c in range(4):
        out[35 + c] = q[c] * qinv

    for c in range(39):
        e_ref[c, 0] = out[c]


def _features(Gn, table, Dnb, Eidx):
    B, L, K = Dnb.shape
    gn = Gn.reshape(_NCOMP, B, L, K)
    S = L // _FROWS
    return pl.pallas_call(
        _features_body,
        grid=(B, S),
        in_specs=[
            pl.BlockSpec((_NCOMP, 1, _FROWS, K), lambda b, s: (0, b, s, 0)),
            pl.BlockSpec((1, _FROWS, 12), lambda b, s: (b, s, 0)),
            pl.BlockSpec((1, _FROWS, K), lambda b, s: (b, s, 0)),
            pl.BlockSpec((1, _FROWS, K), lambda b, s: (b, s, 0)),
        ],
        out_specs=pl.BlockSpec((39, 1, _FROWS, K), lambda b, s: (0, b, s, 0)),
        out_shape=jax.ShapeDtypeStruct((39, B, L, K), jnp.float32),
    )(gn, table, Dnb, Eidx)


def kernel(X, mask):
    B, N = X.shape[0], X.shape[2]
    K = TOP_K
    Xr = X.reshape(B, N, 12)
    Xc = X[:, 0, :, 1, :]

    D_neighbors, E_idx, G_idx = _dist_topk(Xc)
    table = _table_kernel(Xc)

    tableT = table.reshape(B * N, _NCOMP).T
    Gn = _sc_gather(tableT, G_idx.reshape(-1))
    V = _dihedrals_kernel(Xr)
    Eplanes = _features(Gn, table, D_neighbors, E_idx)
    E = jnp.transpose(Eplanes, (1, 2, 3, 0))
    return (V, E, E_idx)
